# baseline (speedup 1.0000x reference)
import functools

import jax
import jax.numpy as jnp
from jax import lax
from jax.experimental import pallas as pl
from jax.experimental.pallas import tpu as pltpu
from jax.experimental.pallas import tpu_sc as plsc

B, S, E = 4, 4096, 1024
L = 16
NC, NS = 2, 16
C = 8
NXB = 3
GSPLIT = (18, 14)

assert GSPLIT[0] + GSPLIT[1] == S // (NS * C)


def kernel(x, pos_table):
    xf = x.reshape(B * S, E)

    mesh = plsc.VectorSubcoreMesh(core_axis_name="c", subcore_axis_name="s")

    scratch = (
        [pltpu.VMEM((C, E), jnp.float32) for _ in range(B * NXB)]
        + [pltpu.VMEM((C, E), jnp.float32) for _ in range(2)]
        + [pltpu.SemaphoreType.DMA for _ in range(B * NXB + 2)]
    )

    @functools.partial(
        pl.kernel,
        out_type=jax.ShapeDtypeStruct((B * S, E), jnp.float32),
        mesh=mesh,
        scratch_types=scratch,
    )
    def sc_add(x_hbm, pos_hbm, out_hbm, *bufs):
        xb = [[bufs[b * NXB + j] for j in range(NXB)] for b in range(B)]
        pb = [bufs[B * NXB], bufs[B * NXB + 1]]
        sems = bufs[B * NXB + 2:]
        sxb = [[sems[b * NXB + j] for j in range(NXB)] for b in range(B)]
        spb = [sems[B * NXB], sems[B * NXB + 1]]

        cid = lax.axis_index("c")
        sid = lax.axis_index("s")

        def pipeline(pos0, ng):
            def p_row(g):
                return pos0 + g * C

            def x_row(b, g):
                return b * S + pos0 + g * C

            def start_x_in(b, g):
                j = g % NXB
                pltpu.async_copy(
                    x_hbm.at[pl.ds(x_row(b, g), C)], xb[b][j], sxb[b][j]
                )

            def start_p_in(g):
                pltpu.async_copy(
                    pos_hbm.at[pl.ds(p_row(g), C)], pb[g % 2], spb[g % 2]
                )

            def wait_x_in(b, g):
                j = g % NXB
                pltpu.make_async_copy(
                    x_hbm.at[pl.ds(x_row(b, g), C)], xb[b][j], sxb[b][j]
                ).wait()

            def wait_out(b, g):
                j = g % NXB
                pltpu.make_async_copy(
                    xb[b][j], out_hbm.at[pl.ds(x_row(b, g), C)], sxb[b][j]
                ).wait()

            start_p_in(0)
            for b in range(B):
                start_x_in(b, 0)

            for g in range(ng):
                if g + 1 < ng:
                    for b in range(B):
                        if g - 2 >= 0:
                            wait_out(b, g - 2)
                        start_x_in(b, g + 1)
                    start_p_in(g + 1)
                for b in range(B):
                    wait_x_in(b, g)
                pltpu.make_async_copy(
                    pos_hbm.at[pl.ds(p_row(g), C)], pb[g % 2], spb[g % 2]
                ).wait()

                xg = [xb[b][g % NXB] for b in range(B)]
                pg = pb[g % 2]

                @plsc.parallel_loop(0, E, step=L, unroll=1)
                def add_body(o):
                    for r in range(C):
                        pv = pg[r, pl.ds(o, L)]
                        for b in range(B):
                            xg[b][r, pl.ds(o, L)] = xg[b][r, pl.ds(o, L)] + pv

                for b in range(B):
                    j = g % NXB
                    pltpu.async_copy(
                        xg[b], out_hbm.at[pl.ds(x_row(b, g), C)], sxb[b][j]
                    )

            for g in range(max(0, ng - 3), ng):
                for b in range(B):
                    wait_out(b, g)

        g0, g1 = GSPLIT

        @pl.when(cid == 0)
        def _():
            pipeline(sid * (g0 * C), g0)

        @pl.when(cid == 1)
        def _():
            pipeline(NS * g0 * C + sid * (g1 * C), g1)

    out = sc_add(xf, pos_table)
    return out.reshape(B, S, E)


# speedup vs baseline: 1.0304x; 1.0304x over previous
import functools

import jax
import jax.numpy as jnp
from jax import lax
from jax.experimental import pallas as pl
from jax.experimental.pallas import tpu as pltpu
from jax.experimental.pallas import tpu_sc as plsc

B, S, E = 4, 4096, 1024
L = 16
NC, NS = 2, 16
C = 8
NXB = 3
GSPLIT = (14, 18)

assert GSPLIT[0] + GSPLIT[1] == S // (NS * C)


def kernel(x, pos_table):
    xf = x.reshape(B * S, E)

    mesh = plsc.VectorSubcoreMesh(core_axis_name="c", subcore_axis_name="s")

    scratch = (
        [pltpu.VMEM((C, E), jnp.float32) for _ in range(B * NXB)]
        + [pltpu.VMEM((C, E), jnp.float32) for _ in range(2)]
        + [pltpu.SemaphoreType.DMA for _ in range(B * NXB + 2)]
    )

    @functools.partial(
        pl.kernel,
        out_type=jax.ShapeDtypeStruct((B * S, E), jnp.float32),
        mesh=mesh,
        scratch_types=scratch,
    )
    def sc_add(x_hbm, pos_hbm, out_hbm, *bufs):
        xb = [[bufs[b * NXB + j] for j in range(NXB)] for b in range(B)]
        pb = [bufs[B * NXB], bufs[B * NXB + 1]]
        sems = bufs[B * NXB + 2:]
        sxb = [[sems[b * NXB + j] for j in range(NXB)] for b in range(B)]
        spb = [sems[B * NXB], sems[B * NXB + 1]]

        cid = lax.axis_index("c")
        sid = lax.axis_index("s")

        def pipeline(pos0, ng):
            def p_row(g):
                return pos0 + g * C

            def x_row(b, g):
                return b * S + pos0 + g * C

            def start_x_in(b, g):
                j = g % NXB
                pltpu.async_copy(
                    x_hbm.at[pl.ds(x_row(b, g), C)], xb[b][j], sxb[b][j]
                )

            def start_p_in(g):
                pltpu.async_copy(
                    pos_hbm.at[pl.ds(p_row(g), C)], pb[g % 2], spb[g % 2]
                )

            def wait_x_in(b, g):
                j = g % NXB
                pltpu.make_async_copy(
                    x_hbm.at[pl.ds(x_row(b, g), C)], xb[b][j], sxb[b][j]
                ).wait()

            def wait_out(b, g):
                j = g % NXB
                pltpu.make_async_copy(
                    xb[b][j], out_hbm.at[pl.ds(x_row(b, g), C)], sxb[b][j]
                ).wait()

            start_p_in(0)
            for b in range(B):
                start_x_in(b, 0)

            for g in range(ng):
                if g + 1 < ng:
                    for b in range(B):
                        if g - 2 >= 0:
                            wait_out(b, g - 2)
                        start_x_in(b, g + 1)
                    start_p_in(g + 1)
                for b in range(B):
                    wait_x_in(b, g)
                pltpu.make_async_copy(
                    pos_hbm.at[pl.ds(p_row(g), C)], pb[g % 2], spb[g % 2]
                ).wait()

                xg = [xb[b][g % NXB] for b in range(B)]
                pg = pb[g % 2]

                @plsc.parallel_loop(0, E, step=L, unroll=1)
                def add_body(o):
                    for r in range(C):
                        pv = pg[r, pl.ds(o, L)]
                        for b in range(B):
                            xg[b][r, pl.ds(o, L)] = xg[b][r, pl.ds(o, L)] + pv

                for b in range(B):
                    j = g % NXB
                    pltpu.async_copy(
                        xg[b], out_hbm.at[pl.ds(x_row(b, g), C)], sxb[b][j]
                    )

            for g in range(max(0, ng - 3), ng):
                for b in range(B):
                    wait_out(b, g)

        g0, g1 = GSPLIT

        @pl.when(cid == 0)
        def _():
            pipeline(sid * (g0 * C), g0)

        @pl.when(cid == 1)
        def _():
            pipeline(NS * g0 * C + sid * (g1 * C), g1)

    out = sc_add(xf, pos_table)
    return out.reshape(B, S, E)


# speedup vs baseline: 1.1308x; 1.0974x over previous
import functools

import jax
import jax.numpy as jnp
from jax import lax
from jax.experimental import pallas as pl
from jax.experimental.pallas import tpu as pltpu
from jax.experimental.pallas import tpu_sc as plsc

B, S, E = 4, 4096, 1024
L = 16
NC, NS = 2, 16
NW = NC * NS
PPW = S // NW
C = 8
NG = PPW // C
NXB = 3


def kernel(x, pos_table):
    xf = x.reshape(B * S, E)

    mesh = plsc.VectorSubcoreMesh(core_axis_name="c", subcore_axis_name="s")

    scratch = (
        [pltpu.VMEM((C, E), jnp.float32) for _ in range(B * NXB)]
        + [pltpu.VMEM((C, E), jnp.float32) for _ in range(2)]
        + [pltpu.SemaphoreType.DMA for _ in range(B * NXB + 2)]
    )

    @functools.partial(
        pl.kernel,
        out_type=jax.ShapeDtypeStruct((B * S, E), jnp.float32),
        mesh=mesh,
        scratch_types=scratch,
    )
    def sc_add(x_hbm, pos_hbm, out_hbm, *bufs):
        xb = [[bufs[b * NXB + j] for j in range(NXB)] for b in range(B)]
        pb = [bufs[B * NXB], bufs[B * NXB + 1]]
        sems = bufs[B * NXB + 2:]
        sxb = [[sems[b * NXB + j] for j in range(NXB)] for b in range(B)]
        spb = [sems[B * NXB], sems[B * NXB + 1]]

        cid = lax.axis_index("c")
        sid = lax.axis_index("s")

        def pipeline(pos0, ng):
            def p_row(g):
                return pos0 + g * C

            def x_row(b, g):
                return b * S + pos0 + g * C

            def start_x_in(b, g):
                j = g % NXB
                pltpu.async_copy(
                    x_hbm.at[pl.ds(x_row(b, g), C)], xb[b][j], sxb[b][j]
                )

            def start_p_in(g):
                pltpu.async_copy(
                    pos_hbm.at[pl.ds(p_row(g), C)], pb[g % 2], spb[g % 2]
                )

            def wait_x_in(b, g):
                j = g % NXB
                pltpu.make_async_copy(
                    x_hbm.at[pl.ds(x_row(b, g), C)], xb[b][j], sxb[b][j]
                ).wait()

            def wait_out(b, g):
                j = g % NXB
                pltpu.make_async_copy(
                    xb[b][j], out_hbm.at[pl.ds(x_row(b, g), C)], sxb[b][j]
                ).wait()

            start_p_in(0)
            for b in range(B):
                start_x_in(b, 0)

            for g in range(ng):
                if g + 1 < ng:
                    for b in range(B):
                        if g - 2 >= 0:
                            wait_out(b, g - 2)
                        start_x_in(b, g + 1)
                    start_p_in(g + 1)
                for b in range(B):
                    wait_x_in(b, g)
                pltpu.make_async_copy(
                    pos_hbm.at[pl.ds(p_row(g), C)], pb[g % 2], spb[g % 2]
                ).wait()

                xg = [xb[b][g % NXB] for b in range(B)]
                pg = pb[g % 2]

                @plsc.parallel_loop(0, E, step=L, unroll=1)
                def add_body(o):
                    for r in range(C):
                        pv = pg[r, pl.ds(o, L)]
                        for b in range(B):
                            xg[b][r, pl.ds(o, L)] = xg[b][r, pl.ds(o, L)] + pv

                for b in range(B):
                    j = g % NXB
                    pltpu.async_copy(
                        xg[b], out_hbm.at[pl.ds(x_row(b, g), C)], sxb[b][j]
                    )

            for g in range(max(0, ng - 3), ng):
                for b in range(B):
                    wait_out(b, g)

        wid = sid * NC + cid
        pipeline(wid * PPW, NG)

    out = sc_add(xf, pos_table)
    return out.reshape(B, S, E)
